# baseline (speedup 1.0000x reference)
import jax
import jax.numpy as jnp
from jax.experimental import pallas as pl
import jax.experimental.pallas.tpu as pltpu

_B = 16
_C = 768
_P = 512
_HW = 1024


def _concat_pos_kernel(x_hbm, row_ref, col_ref, o_hbm, pos_vmem, sem):
    col_t = col_ref[...].T
    row_t = row_ref[...].T
    pos_vmem[:256, :] = jnp.broadcast_to(col_t[:, None, :], (256, 32, 32)).reshape(256, _HW)
    pos_vmem[256:, :] = jnp.broadcast_to(row_t[:, :, None], (256, 32, 32)).reshape(256, _HW)

    copies = []
    for b in range(_B):
        copies.append(pltpu.make_async_copy(
            x_hbm.at[b], o_hbm.at[b, 0:_C], sem))
        copies.append(pltpu.make_async_copy(
            pos_vmem, o_hbm.at[b, _C:_C + _P], sem))
    for c in copies:
        c.start()
    for c in copies:
        c.wait()


def kernel(x, row_embed, col_embed):
    b, c, h, w = x.shape
    x2 = x.reshape(b, c, h * w)
    out = pl.pallas_call(
        _concat_pos_kernel,
        in_specs=[
            pl.BlockSpec(memory_space=pl.ANY),
            pl.BlockSpec(memory_space=pltpu.MemorySpace.VMEM),
            pl.BlockSpec(memory_space=pltpu.MemorySpace.VMEM),
        ],
        out_specs=pl.BlockSpec(memory_space=pl.ANY),
        out_shape=jax.ShapeDtypeStruct((b, c + _P, h * w), x.dtype),
        scratch_shapes=[
            pltpu.VMEM((_P, h * w), x.dtype),
            pltpu.SemaphoreType.DMA,
        ],
    )(x2, row_embed, col_embed)
    return out.reshape(b, c + _P, h, w)


# speedup vs baseline: 9.9741x; 9.9741x over previous
import jax
import jax.numpy as jnp
from jax.experimental import pallas as pl
import jax.experimental.pallas.tpu as pltpu

_B = 16
_C = 768
_P = 512
_HW = 1024
_SPLIT = 2
_CH = _C // _SPLIT
_NCHUNK = _B * _SPLIT
_K = 12
_W = 6


def _concat_pos_kernel(x_hbm, row_ref, col_ref, o_hbm, stage, pos_vmem,
                       in_sems, out_sems, pos_sem):
    col_t = col_ref[...].T
    row_t = row_ref[...].T
    pos_vmem[:256, :] = jnp.broadcast_to(col_t[:, None, :], (256, 32, 32)).reshape(256, _HW)
    pos_vmem[256:, :] = jnp.broadcast_to(row_t[:, :, None], (256, 32, 32)).reshape(256, _HW)

    pos_copies = [
        pltpu.make_async_copy(pos_vmem, o_hbm.at[b, _C:_C + _P], pos_sem)
        for b in range(_B)
    ]
    for cp in pos_copies:
        cp.start()

    def in_copy(i):
        b, half = divmod(i, _SPLIT)
        return pltpu.make_async_copy(
            x_hbm.at[b, half * _CH:(half + 1) * _CH],
            stage.at[i % _K], in_sems.at[i % _K])

    def out_copy(i):
        b, half = divmod(i, _SPLIT)
        return pltpu.make_async_copy(
            stage.at[i % _K],
            o_hbm.at[b, half * _CH:(half + 1) * _CH], out_sems.at[i % _K])

    out_copies = [out_copy(i) for i in range(_NCHUNK)]
    for i in range(_K):
        in_copy(i).start()
    for i in range(_NCHUNK):
        in_copy(i).wait()
        out_copies[i].start()
        j = i - _W
        if j >= 0 and j + _K < _NCHUNK:
            out_copies[j].wait()
            in_copy(j + _K).start()
    for i in range(_NCHUNK - _K, _NCHUNK):
        out_copies[i].wait()
    for cp in pos_copies:
        cp.wait()


def kernel(x, row_embed, col_embed):
    b, c, h, w = x.shape
    x2 = x.reshape(b, c, h * w)
    out = pl.pallas_call(
        _concat_pos_kernel,
        in_specs=[
            pl.BlockSpec(memory_space=pl.ANY),
            pl.BlockSpec(memory_space=pltpu.MemorySpace.VMEM),
            pl.BlockSpec(memory_space=pltpu.MemorySpace.VMEM),
        ],
        out_specs=pl.BlockSpec(memory_space=pl.ANY),
        out_shape=jax.ShapeDtypeStruct((b, c + _P, h * w), x.dtype),
        scratch_shapes=[
            pltpu.VMEM((_K, _CH, h * w), x.dtype),
            pltpu.VMEM((_P, h * w), x.dtype),
            pltpu.SemaphoreType.DMA((_K,)),
            pltpu.SemaphoreType.DMA((_K,)),
            pltpu.SemaphoreType.DMA,
        ],
    )(x2, row_embed, col_embed)
    return out.reshape(b, c + _P, h, w)


# speedup vs baseline: 38.7049x; 3.8805x over previous
import jax
import jax.numpy as jnp
from jax.experimental import pallas as pl
import jax.experimental.pallas.tpu as pltpu

_B = 16
_C = 768
_P = 512
_HW = 1024
_K = 8
_W = 4


def _concat_pos_kernel(x_hbm, row_ref, col_ref, o_hbm, stage, in_sems, out_sems):
    colb = jnp.broadcast_to(col_ref[...][None, :, :], (32, 32, 256)).reshape(_HW, 256)
    rowb = jnp.broadcast_to(row_ref[...][:, None, :], (32, 32, 256)).reshape(_HW, 256)
    for s in range(_K):
        stage[s, :, _C:_C + 256] = colb
        stage[s, :, _C + 256:] = rowb

    def in_copy(i):
        return pltpu.make_async_copy(
            x_hbm.at[i], stage.at[i % _K, :, 0:_C], in_sems.at[i % _K])

    out_copies = [
        pltpu.make_async_copy(stage.at[i % _K], o_hbm.at[i], out_sems.at[i % _K])
        for i in range(_B)
    ]

    for i in range(_K):
        in_copy(i).start()
    for i in range(_B):
        in_copy(i).wait()
        out_copies[i].start()
        j = i - _W
        if j >= 0 and j + _K < _B:
            out_copies[j].wait()
            in_copy(j + _K).start()
    for i in range(_B - _K, _B):
        out_copies[i].wait()


def kernel(x, row_embed, col_embed):
    b, c, h, w = x.shape
    xt = x.transpose(0, 2, 3, 1).reshape(b, h * w, c)
    out = pl.pallas_call(
        _concat_pos_kernel,
        in_specs=[
            pl.BlockSpec(memory_space=pl.ANY),
            pl.BlockSpec(memory_space=pltpu.MemorySpace.VMEM),
            pl.BlockSpec(memory_space=pltpu.MemorySpace.VMEM),
        ],
        out_specs=pl.BlockSpec(memory_space=pl.ANY),
        out_shape=jax.ShapeDtypeStruct((b, h * w, c + _P), x.dtype),
        scratch_shapes=[
            pltpu.VMEM((_K, h * w, c + _P), x.dtype),
            pltpu.SemaphoreType.DMA((_K,)),
            pltpu.SemaphoreType.DMA((_K,)),
        ],
    )(xt, row_embed, col_embed)
    return out.reshape(b, h, w, c + _P).transpose(0, 3, 1, 2)
